# Initial kernel scaffold; baseline (speedup 1.0000x reference)
import jax
import jax.numpy as jnp
from jax.experimental import pallas as pl


def kernel(x, edge_index, batch, W1a, b1a, W1b, b1b, W2a, b2a, W2b, b2b, g1, be1, g2, be2, Wf1, bf1, Wf2, bf2):
    raise NotImplementedError("write your pallas kernel here")


# speedup vs baseline: 4.6294x; 4.6294x over previous
import functools

import jax
import jax.numpy as jnp
from jax import lax
from jax.experimental import pallas as pl
from jax.experimental.pallas import tpu as pltpu
from jax.experimental.pallas import tpu_sc as plsc

HI = jax.lax.Precision.HIGHEST

N = 10000
E = 320000
NFEAT = 128
HDIM = 32
NGRAPH = 64

NW = 32
CH = 128
EPW = 10240
EP = NW * EPW
KCH = EPW // CH
NPAD = 10112
STRIPE = NPAD // 16
EPS = 1e-5


def _b16(a):
  return a.astype(jnp.bfloat16).astype(jnp.float32)


def _sc_edge_aggregate(feats, src_rows, dst_rows, zeros_pad, f):
  mesh = plsc.VectorSubcoreMesh(core_axis_name="c", subcore_axis_name="s")

  @functools.partial(
      pl.kernel,
      out_type=jax.ShapeDtypeStruct((2, NPAD, f), jnp.float32),
      mesh=mesh,
      compiler_params=pltpu.CompilerParams(use_tc_tiling_on_sc=False),
      scratch_types=[
          pltpu.VMEM((KCH, CH), jnp.int32),
          pltpu.VMEM((KCH, CH), jnp.int32),
          pltpu.VMEM((CH, f), jnp.float32),
          pltpu.VMEM_SHARED((NPAD, f), jnp.float32),
          pltpu.SemaphoreType.DMA,
      ],
  )
  def k(y_hbm, src_hbm, dst_hbm, z_hbm, out_hbm, idx_s, idx_d, gbuf, acc, gsem):
    c = lax.axis_index("c")
    s = lax.axis_index("s")
    wid = s * 2 + c
    pltpu.sync_copy(z_hbm.at[pl.ds(s * STRIPE, STRIPE)],
                    acc.at[pl.ds(s * STRIPE, STRIPE)])
    pltpu.sync_copy(src_hbm.at[pl.ds(wid * KCH, KCH)], idx_s)
    pltpu.sync_copy(dst_hbm.at[pl.ds(wid * KCH, KCH)], idx_d)
    plsc.subcore_barrier()

    def body(j, carry):
      pltpu.async_copy(y_hbm.at[idx_s.at[j]], gbuf, gsem).wait()
      pltpu.sync_copy(gbuf, acc.at[idx_d.at[j]], add=True)
      return carry

    lax.fori_loop(0, KCH, body, 0)
    plsc.subcore_barrier()
    pltpu.sync_copy(acc.at[pl.ds(s * STRIPE, STRIPE)],
                    out_hbm.at[c, pl.ds(s * STRIPE, STRIPE)])

  return k(feats, src_rows, dst_rows, zeros_pad)


def _tc_layer1(x, parts, w1a, b1a, w1b, b1b, g1, be1):
  def body(x_ref, p_ref, w1a_ref, b1a_ref, w1b_ref, b1b_ref, g1_ref,
           be1_ref, hbn_ref):
    z = x_ref[...] + p_ref[0, :N, :] + p_ref[1, :N, :]
    t = _b16(jnp.maximum(
        jnp.dot(z, w1a_ref[...], preferred_element_type=jnp.float32)
        + b1a_ref[...], 0.0))
    h1 = jnp.maximum(
        jnp.dot(t, w1b_ref[...], preferred_element_type=jnp.float32)
        + b1b_ref[...], 0.0)
    m = jnp.sum(h1, axis=0, keepdims=True) * (1.0 / N)
    v = jnp.sum((h1 - m) ** 2, axis=0, keepdims=True) * (1.0 / N)
    hbn_ref[...] = (h1 - m) / jnp.sqrt(v + EPS) * g1_ref[...] + be1_ref[...]

  return pl.pallas_call(
      body,
      out_shape=jax.ShapeDtypeStruct((N, HDIM), jnp.float32),
  )(x, parts, w1a, b1a, w1b, b1b, g1, be1)


def _tc_layer2(hbn, parts, w2a, b2a, w2b, b2b, g2, be2, batch_col,
               wf1, bf1, wf2, bf2):
  def body(h_ref, p_ref, w2a_ref, b2a_ref, w2b_ref, b2b_ref, g2_ref,
           be2_ref, bat_ref, wf1_ref, bf1_ref, wf2_ref, bf2_ref,
           emb_ref, sc_ref):
    z = h_ref[...] + p_ref[0, :N, :] + p_ref[1, :N, :]
    t = _b16(jnp.maximum(
        jnp.dot(z, w2a_ref[...], preferred_element_type=jnp.float32)
        + b2a_ref[...], 0.0))
    h2 = jnp.maximum(
        jnp.dot(t, w2b_ref[...], preferred_element_type=jnp.float32)
        + b2b_ref[...], 0.0)
    m = jnp.sum(h2, axis=0, keepdims=True) * (1.0 / N)
    v = jnp.sum((h2 - m) ** 2, axis=0, keepdims=True) * (1.0 / N)
    h2bn = (h2 - m) / jnp.sqrt(v + EPS) * g2_ref[...] + be2_ref[...]
    ids = lax.broadcasted_iota(jnp.int32, (N, NGRAPH), 1)
    oh = (ids == bat_ref[...]).astype(jnp.float32)
    pooled = lax.dot_general(oh, h2bn, (((0,), (0,)), ((), ())),
                             preferred_element_type=jnp.float32, precision=HI)
    e = jnp.maximum(
        jnp.dot(pooled, wf1_ref[...], preferred_element_type=jnp.float32, precision=HI)
        + bf1_ref[...], 0.0)
    emb_ref[...] = e
    sc_ref[...] = jnp.dot(e, wf2_ref[...],
                          preferred_element_type=jnp.float32, precision=HI) + bf2_ref[...]

  return pl.pallas_call(
      body,
      out_shape=(jax.ShapeDtypeStruct((NGRAPH, HDIM), jnp.float32),
                 jax.ShapeDtypeStruct((NGRAPH, 1), jnp.float32)),
  )(hbn, parts, w2a, b2a, w2b, b2b, g2, be2, batch_col, wf1, bf1, wf2, bf2)


def kernel(x, edge_index, batch, W1a, b1a, W1b, b1b, W2a, b2a, W2b, b2b,
           g1, be1, g2, be2, Wf1, bf1, Wf2, bf2):
  src = edge_index[0]
  dst = edge_index[1]
  pad = EP - E
  src_rows = jnp.concatenate(
      [src, jnp.zeros((pad,), jnp.int32)]).reshape(EP // CH, CH)
  dst_rows = jnp.concatenate(
      [dst, jnp.full((pad,), N, jnp.int32)]).reshape(EP // CH, CH)
  zeros_nf = jnp.zeros((NPAD, NFEAT), jnp.float32)
  zeros_h = jnp.zeros((NPAD, HDIM), jnp.float32)
  batch_col = batch.reshape(N, 1)
  r = lambda a: a.reshape(1, -1)

  parts1 = _sc_edge_aggregate(x, src_rows, dst_rows, zeros_nf, NFEAT)
  hbn = _tc_layer1(x, parts1, W1a, r(b1a), W1b, r(b1b), r(g1), r(be1))
  parts2 = _sc_edge_aggregate(hbn, src_rows, dst_rows, zeros_h, HDIM)
  emb, score = _tc_layer2(hbn, parts2, W2a, r(b2a), W2b, r(b2b), r(g2),
                          r(be2), batch_col, Wf1, r(bf1), Wf2, r(bf2))
  return (emb, score)
